# baseline (speedup 1.0000x reference)
import functools

import jax
import jax.numpy as jnp
from jax import lax
from jax.experimental import pallas as pl
from jax.experimental.pallas import tpu as pltpu
from jax.experimental.pallas import tpu_sc as plsc

N_NODES = 10000
N_EDGES = 320000
N_GRAPHS = 64
D = 128
HID = 256
NUM_LAYERS = 5

NC = 2
NS = 16
NW = NC * NS

EDGES_PER_W = N_EDGES // NW
CHUNK = 128
NCHUNK = 80
EDGES_PAD = NCHUNK * CHUNK

ROWS_PAD = 10240
RPW = ROWS_PAD // NS
DUMMY_ROW = N_NODES

ROW_BLK = 1000
N_BLOCKS = N_NODES // ROW_BLK


def _make_sc_agg():
    mesh = plsc.VectorSubcoreMesh(core_axis_name="c", subcore_axis_name="s")

    @functools.partial(
        pl.kernel,
        mesh=mesh,
        out_type=jax.ShapeDtypeStruct((NC, ROWS_PAD, D), jnp.float32),
        scratch_types=[
            pltpu.VMEM((NCHUNK // 2, CHUNK), jnp.int32),
            pltpu.VMEM((NCHUNK // 2, CHUNK), jnp.int32),
            pltpu.VMEM((CHUNK, D), jnp.float32),
            pltpu.VMEM((CHUNK, D), jnp.float32),
            pltpu.VMEM_SHARED((ROWS_PAD, D), jnp.float32),
            pltpu.SemaphoreType.DMA,
            pltpu.SemaphoreType.DMA,
        ],
    )
    def sc_agg(h_hbm, srcw_hbm, dstw_hbm, zeros_hbm, out_hbm,
               src_v, dst_v, buf_a, buf_b, acc, sem_a, sem_b):
        c = lax.axis_index("c")
        s = lax.axis_index("s")
        wid = c * NS + s

        pltpu.sync_copy(zeros_hbm.at[pl.ds(s * RPW, RPW)],
                        acc.at[pl.ds(s * RPW, RPW)])
        plsc.subcore_barrier()

        cpp = NCHUNK // 2
        for p in range(2):
            pltpu.sync_copy(srcw_hbm.at[wid, pl.ds(p * cpp, cpp)], src_v)
            pltpu.sync_copy(dstw_hbm.at[wid, pl.ds(p * cpp, cpp)], dst_v)

            @pl.loop(0, cpp, step=2)
            def _(j):
                ga = pltpu.async_copy(h_hbm.at[src_v.at[j]], buf_a, sem_a)
                gb = pltpu.async_copy(h_hbm.at[src_v.at[j + 1]], buf_b, sem_b)
                ga.wait()
                pltpu.sync_copy(buf_a, acc.at[dst_v.at[j]], add=True)
                gb.wait()
                pltpu.sync_copy(buf_b, acc.at[dst_v.at[j + 1]], add=True)

        plsc.subcore_barrier()
        pltpu.sync_copy(acc.at[pl.ds(s * RPW, RPW)],
                        out_hbm.at[c, pl.ds(s * RPW, RPW)])

    return sc_agg


_sc_agg = _make_sc_agg()


def _mid_body(h_ref, parts_ref, w1_ref, b1_ref, w2_ref, b2_ref,
              eps_ref, ws_ref, out_ref):
    m = h_ref[...] * (1.0 + eps_ref[...]) + parts_ref[0] + parts_ref[1]
    t = jnp.dot(m, w1_ref[...], preferred_element_type=jnp.float32)
    t = jnp.maximum(t + b1_ref[...], 0.0)
    n = jnp.dot(t, w2_ref[...], preferred_element_type=jnp.float32)
    n = jnp.maximum(n + b2_ref[...], 0.0)
    score = jnp.tanh(jnp.dot(n, ws_ref[...], preferred_element_type=jnp.float32))
    out_ref[...] = n * score


def _tc_mid(h, parts, w1, b1, w2, b2, epsl, ws_col):
    return pl.pallas_call(
        _mid_body,
        grid=(N_BLOCKS,),
        in_specs=[
            pl.BlockSpec((ROW_BLK, D), lambda i: (i, 0)),
            pl.BlockSpec((NC, ROW_BLK, D), lambda i: (0, i, 0)),
            pl.BlockSpec((D, HID), lambda i: (0, 0)),
            pl.BlockSpec((1, HID), lambda i: (0, 0)),
            pl.BlockSpec((HID, D), lambda i: (0, 0)),
            pl.BlockSpec((1, D), lambda i: (0, 0)),
            pl.BlockSpec((1, 1), lambda i: (0, 0)),
            pl.BlockSpec((D, 1), lambda i: (0, 0)),
        ],
        out_specs=pl.BlockSpec((ROW_BLK, D), lambda i: (i, 0)),
        out_shape=jax.ShapeDtypeStruct((N_NODES, D), jnp.float32),
    )(h, parts, w1, b1, w2, b2, epsl, ws_col)


def _last_body(h_ref, parts_ref, w1_ref, b1_ref, w2_ref, b2_ref,
               eps_ref, ws_ref, batch_ref, wp_ref, bp_ref,
               pooled_ref, out_ref):
    i = pl.program_id(0)
    m = h_ref[...] * (1.0 + eps_ref[...]) + parts_ref[0] + parts_ref[1]
    t = jnp.dot(m, w1_ref[...], preferred_element_type=jnp.float32)
    t = jnp.maximum(t + b1_ref[...], 0.0)
    n = jnp.dot(t, w2_ref[...], preferred_element_type=jnp.float32)
    n = n + b2_ref[...]
    score = jnp.tanh(jnp.dot(n, ws_ref[...], preferred_element_type=jnp.float32))
    hout = n * score
    bt = jnp.reshape(batch_ref[...], (1, ROW_BLK))
    onehot_t = (lax.broadcasted_iota(jnp.int32, (N_GRAPHS, ROW_BLK), 0)
                == bt).astype(jnp.float32)
    contrib = jnp.dot(onehot_t, hout, preferred_element_type=jnp.float32,
                      precision=lax.Precision.HIGHEST)

    @pl.when(i == 0)
    def _():
        pooled_ref[...] = contrib

    @pl.when(i != 0)
    def _():
        pooled_ref[...] += contrib

    @pl.when(i == N_BLOCKS - 1)
    def _():
        hg = pooled_ref[...]
        out_ref[...] = (jnp.dot(hg, wp_ref[...],
                                preferred_element_type=jnp.float32)
                        + bp_ref[...])


def _tc_last(h, parts, w1, b1, w2, b2, epsl, ws_col, batch3, wp_row, bp2):
    pooled, out = pl.pallas_call(
        _last_body,
        grid=(N_BLOCKS,),
        in_specs=[
            pl.BlockSpec((ROW_BLK, D), lambda i: (i, 0)),
            pl.BlockSpec((NC, ROW_BLK, D), lambda i: (0, i, 0)),
            pl.BlockSpec((D, HID), lambda i: (0, 0)),
            pl.BlockSpec((1, HID), lambda i: (0, 0)),
            pl.BlockSpec((HID, D), lambda i: (0, 0)),
            pl.BlockSpec((1, D), lambda i: (0, 0)),
            pl.BlockSpec((1, 1), lambda i: (0, 0)),
            pl.BlockSpec((D, 1), lambda i: (0, 0)),
            pl.BlockSpec((1, 1, ROW_BLK), lambda i: (i, 0, 0)),
            pl.BlockSpec((D, 1), lambda i: (0, 0)),
            pl.BlockSpec((1, 1), lambda i: (0, 0)),
        ],
        out_specs=[
            pl.BlockSpec((N_GRAPHS, D), lambda i: (0, 0)),
            pl.BlockSpec((N_GRAPHS, 1), lambda i: (0, 0)),
        ],
        out_shape=[
            jax.ShapeDtypeStruct((N_GRAPHS, D), jnp.float32),
            jax.ShapeDtypeStruct((N_GRAPHS, 1), jnp.float32),
        ],
    )(h, parts, w1, b1, w2, b2, epsl, ws_col, batch3, wp_row, bp2)
    del pooled
    return out


def kernel(x, edge_index, batch, W1, b1, W2, b2, eps, w_score, Wp, bp):
    src = edge_index[0]
    dst = edge_index[1]

    srcw = jnp.pad(src.reshape(NW, EDGES_PER_W),
                   ((0, 0), (0, EDGES_PAD - EDGES_PER_W)),
                   constant_values=0).reshape(NW, NCHUNK, CHUNK)
    pad_dst = jnp.broadcast_to(
        DUMMY_ROW + jnp.arange(NW, dtype=jnp.int32)[:, None],
        (NW, EDGES_PAD - EDGES_PER_W))
    dstw = jnp.concatenate(
        [dst.reshape(NW, EDGES_PER_W), pad_dst], axis=1
    ).reshape(NW, NCHUNK, CHUNK)
    zeros = jnp.zeros((ROWS_PAD, D), jnp.float32)
    batch3 = batch.reshape(N_BLOCKS, 1, ROW_BLK)
    wp_row = Wp
    bp2 = bp.reshape(1, 1)

    h = x
    for l in range(NUM_LAYERS):
        parts = _sc_agg(h, srcw, dstw, zeros)
        w1 = W1[l]
        b1l = b1[l].reshape(1, HID)
        w2 = W2[l]
        b2l = b2[l].reshape(1, D)
        epsl = eps[l].reshape(1, 1)
        ws_col = w_score[l].reshape(D, 1)
        if l < NUM_LAYERS - 1:
            h = _tc_mid(h, parts, w1, b1l, w2, b2l, epsl, ws_col)
        else:
            out = _tc_last(h, parts, w1, b1l, w2, b2l, epsl, ws_col,
                           batch3, wp_row, bp2)
    return out


# speedup vs baseline: 1.4118x; 1.4118x over previous
import functools

import jax
import jax.numpy as jnp
from jax import lax
from jax.experimental import pallas as pl
from jax.experimental.pallas import tpu as pltpu
from jax.experimental.pallas import tpu_sc as plsc

N_NODES = 10000
N_EDGES = 320000
N_GRAPHS = 64
D = 128
HID = 256
NUM_LAYERS = 5

NC = 2
NS = 16
NW = NC * NS

EDGES_PER_W = N_EDGES // NW
CHUNK = 128
NCHUNK = 79
EDGES_PAD = NCHUNK * CHUNK

ROWS_PAD = 10240
RPW = ROWS_PAD // NS
DUMMY_ROW = N_NODES

ROW_BLK = 1000
N_BLOCKS = N_NODES // ROW_BLK


def _make_sc_agg():
    mesh = plsc.VectorSubcoreMesh(core_axis_name="c", subcore_axis_name="s")

    @functools.partial(
        pl.kernel,
        mesh=mesh,
        out_type=jax.ShapeDtypeStruct((NC, ROWS_PAD, D), jnp.float32),
        scratch_types=[
            pltpu.VMEM((NCHUNK, CHUNK), jnp.int32),
            pltpu.VMEM((NCHUNK, CHUNK), jnp.int32),
            pltpu.VMEM((CHUNK, D), jnp.float32),
            pltpu.VMEM_SHARED((ROWS_PAD, D), jnp.float32),
            pltpu.SemaphoreType.DMA,
        ],
    )
    def sc_agg(h_hbm, srcw_hbm, dstw_hbm, zeros_hbm, out_hbm,
               src_v, dst_v, rows_v, acc, sem):
        c = lax.axis_index("c")
        s = lax.axis_index("s")
        wid = c * NS + s

        pltpu.sync_copy(zeros_hbm.at[pl.ds(s * RPW, RPW)],
                        acc.at[pl.ds(s * RPW, RPW)])
        plsc.subcore_barrier()

        pltpu.sync_copy(srcw_hbm.at[wid], src_v)
        pltpu.sync_copy(dstw_hbm.at[wid], dst_v)

        @pl.loop(0, NCHUNK)
        def _(j):
            pltpu.async_copy(h_hbm.at[src_v.at[j]], rows_v, sem).wait()
            pltpu.sync_copy(rows_v, acc.at[dst_v.at[j]], add=True)

        plsc.subcore_barrier()
        pltpu.sync_copy(acc.at[pl.ds(s * RPW, RPW)],
                        out_hbm.at[c, pl.ds(s * RPW, RPW)])

    return sc_agg


_sc_agg = _make_sc_agg()


def _mid_body(h_ref, parts_ref, w1_ref, b1_ref, w2_ref, b2_ref,
              eps_ref, ws_ref, out_ref):
    m = h_ref[...] * (1.0 + eps_ref[...]) + parts_ref[0] + parts_ref[1]
    t = jnp.dot(m, w1_ref[...], preferred_element_type=jnp.float32)
    t = jnp.maximum(t + b1_ref[...], 0.0)
    n = jnp.dot(t, w2_ref[...], preferred_element_type=jnp.float32)
    n = jnp.maximum(n + b2_ref[...], 0.0)
    score = jnp.tanh(jnp.dot(n, ws_ref[...], preferred_element_type=jnp.float32))
    out_ref[...] = n * score


def _tc_mid(h, parts, w1, b1, w2, b2, epsl, ws_col):
    return pl.pallas_call(
        _mid_body,
        grid=(N_BLOCKS,),
        in_specs=[
            pl.BlockSpec((ROW_BLK, D), lambda i: (i, 0)),
            pl.BlockSpec((NC, ROW_BLK, D), lambda i: (0, i, 0)),
            pl.BlockSpec((D, HID), lambda i: (0, 0)),
            pl.BlockSpec((1, HID), lambda i: (0, 0)),
            pl.BlockSpec((HID, D), lambda i: (0, 0)),
            pl.BlockSpec((1, D), lambda i: (0, 0)),
            pl.BlockSpec((1, 1), lambda i: (0, 0)),
            pl.BlockSpec((D, 1), lambda i: (0, 0)),
        ],
        out_specs=pl.BlockSpec((ROW_BLK, D), lambda i: (i, 0)),
        out_shape=jax.ShapeDtypeStruct((N_NODES, D), jnp.float32),
    )(h, parts, w1, b1, w2, b2, epsl, ws_col)


def _last_body(h_ref, parts_ref, w1_ref, b1_ref, w2_ref, b2_ref,
               eps_ref, ws_ref, batch_ref, wp_ref, bp_ref,
               pooled_ref, out_ref):
    i = pl.program_id(0)
    m = h_ref[...] * (1.0 + eps_ref[...]) + parts_ref[0] + parts_ref[1]
    t = jnp.dot(m, w1_ref[...], preferred_element_type=jnp.float32)
    t = jnp.maximum(t + b1_ref[...], 0.0)
    n = jnp.dot(t, w2_ref[...], preferred_element_type=jnp.float32)
    n = n + b2_ref[...]
    score = jnp.tanh(jnp.dot(n, ws_ref[...], preferred_element_type=jnp.float32))
    hout = n * score
    bt = jnp.reshape(batch_ref[...], (1, ROW_BLK))
    onehot_t = (lax.broadcasted_iota(jnp.int32, (N_GRAPHS, ROW_BLK), 0)
                == bt).astype(jnp.float32)
    contrib = jnp.dot(onehot_t, hout, preferred_element_type=jnp.float32,
                      precision=lax.Precision.HIGHEST)

    @pl.when(i == 0)
    def _():
        pooled_ref[...] = contrib

    @pl.when(i != 0)
    def _():
        pooled_ref[...] += contrib

    @pl.when(i == N_BLOCKS - 1)
    def _():
        hg = pooled_ref[...]
        out_ref[...] = (jnp.dot(hg, wp_ref[...],
                                preferred_element_type=jnp.float32)
                        + bp_ref[...])


def _tc_last(h, parts, w1, b1, w2, b2, epsl, ws_col, batch3, wp_row, bp2):
    pooled, out = pl.pallas_call(
        _last_body,
        grid=(N_BLOCKS,),
        in_specs=[
            pl.BlockSpec((ROW_BLK, D), lambda i: (i, 0)),
            pl.BlockSpec((NC, ROW_BLK, D), lambda i: (0, i, 0)),
            pl.BlockSpec((D, HID), lambda i: (0, 0)),
            pl.BlockSpec((1, HID), lambda i: (0, 0)),
            pl.BlockSpec((HID, D), lambda i: (0, 0)),
            pl.BlockSpec((1, D), lambda i: (0, 0)),
            pl.BlockSpec((1, 1), lambda i: (0, 0)),
            pl.BlockSpec((D, 1), lambda i: (0, 0)),
            pl.BlockSpec((1, 1, ROW_BLK), lambda i: (i, 0, 0)),
            pl.BlockSpec((D, 1), lambda i: (0, 0)),
            pl.BlockSpec((1, 1), lambda i: (0, 0)),
        ],
        out_specs=[
            pl.BlockSpec((N_GRAPHS, D), lambda i: (0, 0)),
            pl.BlockSpec((N_GRAPHS, 1), lambda i: (0, 0)),
        ],
        out_shape=[
            jax.ShapeDtypeStruct((N_GRAPHS, D), jnp.float32),
            jax.ShapeDtypeStruct((N_GRAPHS, 1), jnp.float32),
        ],
    )(h, parts, w1, b1, w2, b2, epsl, ws_col, batch3, wp_row, bp2)
    del pooled
    return out


def kernel(x, edge_index, batch, W1, b1, W2, b2, eps, w_score, Wp, bp):
    src = edge_index[0]
    dst = edge_index[1]

    srcw = jnp.pad(src.reshape(NW, EDGES_PER_W),
                   ((0, 0), (0, EDGES_PAD - EDGES_PER_W)),
                   constant_values=0).reshape(NW, NCHUNK, CHUNK)
    pad_dst = jnp.broadcast_to(
        DUMMY_ROW + jnp.arange(NW, dtype=jnp.int32)[:, None],
        (NW, EDGES_PAD - EDGES_PER_W))
    dstw = jnp.concatenate(
        [dst.reshape(NW, EDGES_PER_W), pad_dst], axis=1
    ).reshape(NW, NCHUNK, CHUNK)
    zeros = jnp.zeros((ROWS_PAD, D), jnp.float32)
    batch3 = batch.reshape(N_BLOCKS, 1, ROW_BLK)
    wp_row = Wp
    bp2 = bp.reshape(1, 1)

    h = x
    for l in range(NUM_LAYERS):
        parts = _sc_agg(h, srcw, dstw, zeros)
        w1 = W1[l]
        b1l = b1[l].reshape(1, HID)
        w2 = W2[l]
        b2l = b2[l].reshape(1, D)
        epsl = eps[l].reshape(1, 1)
        ws_col = w_score[l].reshape(D, 1)
        if l < NUM_LAYERS - 1:
            h = _tc_mid(h, parts, w1, b1l, w2, b2l, epsl, ws_col)
        else:
            out = _tc_last(h, parts, w1, b1l, w2, b2l, epsl, ws_col,
                           batch3, wp_row, bp2)
    return out


# speedup vs baseline: 1.4241x; 1.0088x over previous
import functools

import jax
import jax.numpy as jnp
from jax import lax
from jax.experimental import pallas as pl
from jax.experimental.pallas import tpu as pltpu
from jax.experimental.pallas import tpu_sc as plsc

N_NODES = 10000
N_EDGES = 320000
N_GRAPHS = 64
D = 128
HID = 256
NUM_LAYERS = 5

NC = 2
NS = 16
NW = NC * NS

EDGES_PER_W = N_EDGES // NW
CHUNK = 128
NCHUNK = 79
EDGES_PAD = NCHUNK * CHUNK

ROWS_PAD = 10240
RPW = ROWS_PAD // NS
DUMMY_ROW = N_NODES

ROW_BLK = 2000
N_BLOCKS = N_NODES // ROW_BLK


def _make_sc_agg():
    mesh = plsc.VectorSubcoreMesh(core_axis_name="c", subcore_axis_name="s")

    @functools.partial(
        pl.kernel,
        mesh=mesh,
        out_type=jax.ShapeDtypeStruct((NC, ROWS_PAD, D), jnp.float32),
        scratch_types=[
            pltpu.VMEM((NCHUNK, CHUNK), jnp.int32),
            pltpu.VMEM((NCHUNK, CHUNK), jnp.int32),
            pltpu.VMEM((CHUNK, D), jnp.float32),
            pltpu.VMEM_SHARED((ROWS_PAD, D), jnp.float32),
            pltpu.SemaphoreType.DMA,
        ],
    )
    def sc_agg(h_hbm, srcw_hbm, dstw_hbm, zeros_hbm, out_hbm,
               src_v, dst_v, rows_v, acc, sem):
        c = lax.axis_index("c")
        s = lax.axis_index("s")
        wid = c * NS + s

        pltpu.sync_copy(zeros_hbm.at[pl.ds(s * RPW, RPW)],
                        acc.at[pl.ds(s * RPW, RPW)])
        plsc.subcore_barrier()

        pltpu.sync_copy(srcw_hbm.at[wid], src_v)
        pltpu.sync_copy(dstw_hbm.at[wid], dst_v)

        @pl.loop(0, NCHUNK)
        def _(j):
            pltpu.async_copy(h_hbm.at[src_v.at[j]], rows_v, sem).wait()
            pltpu.sync_copy(rows_v, acc.at[dst_v.at[j]], add=True)

        plsc.subcore_barrier()
        pltpu.sync_copy(acc.at[pl.ds(s * RPW, RPW)],
                        out_hbm.at[c, pl.ds(s * RPW, RPW)])

    return sc_agg


_sc_agg = _make_sc_agg()


def _mid_body(h_ref, parts_ref, w1_ref, b1_ref, w2_ref, b2_ref,
              eps_ref, ws_ref, out_ref):
    m = h_ref[...] * (1.0 + eps_ref[...]) + parts_ref[0] + parts_ref[1]
    t = jnp.dot(m, w1_ref[...], preferred_element_type=jnp.float32)
    t = jnp.maximum(t + b1_ref[...], 0.0)
    n = jnp.dot(t, w2_ref[...], preferred_element_type=jnp.float32)
    n = jnp.maximum(n + b2_ref[...], 0.0)
    score = jnp.tanh(jnp.dot(n, ws_ref[...], preferred_element_type=jnp.float32))
    out_ref[...] = n * score


def _tc_mid(h, parts, w1, b1, w2, b2, epsl, ws_col):
    return pl.pallas_call(
        _mid_body,
        grid=(N_BLOCKS,),
        in_specs=[
            pl.BlockSpec((ROW_BLK, D), lambda i: (i, 0)),
            pl.BlockSpec((NC, ROW_BLK, D), lambda i: (0, i, 0)),
            pl.BlockSpec((D, HID), lambda i: (0, 0)),
            pl.BlockSpec((1, HID), lambda i: (0, 0)),
            pl.BlockSpec((HID, D), lambda i: (0, 0)),
            pl.BlockSpec((1, D), lambda i: (0, 0)),
            pl.BlockSpec((1, 1), lambda i: (0, 0)),
            pl.BlockSpec((D, 1), lambda i: (0, 0)),
        ],
        out_specs=pl.BlockSpec((ROW_BLK, D), lambda i: (i, 0)),
        out_shape=jax.ShapeDtypeStruct((N_NODES, D), jnp.float32),
    )(h, parts, w1, b1, w2, b2, epsl, ws_col)


def _last_body(h_ref, parts_ref, w1_ref, b1_ref, w2_ref, b2_ref,
               eps_ref, ws_ref, batch_ref, wp_ref, bp_ref,
               pooled_ref, out_ref):
    i = pl.program_id(0)
    m = h_ref[...] * (1.0 + eps_ref[...]) + parts_ref[0] + parts_ref[1]
    t = jnp.dot(m, w1_ref[...], preferred_element_type=jnp.float32)
    t = jnp.maximum(t + b1_ref[...], 0.0)
    n = jnp.dot(t, w2_ref[...], preferred_element_type=jnp.float32)
    n = n + b2_ref[...]
    score = jnp.tanh(jnp.dot(n, ws_ref[...], preferred_element_type=jnp.float32))
    hout = n * score
    bt = jnp.reshape(batch_ref[...], (1, ROW_BLK))
    onehot_t = (lax.broadcasted_iota(jnp.int32, (N_GRAPHS, ROW_BLK), 0)
                == bt).astype(jnp.float32)
    contrib = jnp.dot(onehot_t, hout, preferred_element_type=jnp.float32,
                      precision=lax.Precision.HIGHEST)

    @pl.when(i == 0)
    def _():
        pooled_ref[...] = contrib

    @pl.when(i != 0)
    def _():
        pooled_ref[...] += contrib

    @pl.when(i == N_BLOCKS - 1)
    def _():
        hg = pooled_ref[...]
        out_ref[...] = (jnp.dot(hg, wp_ref[...],
                                preferred_element_type=jnp.float32)
                        + bp_ref[...])


def _tc_last(h, parts, w1, b1, w2, b2, epsl, ws_col, batch3, wp_row, bp2):
    pooled, out = pl.pallas_call(
        _last_body,
        grid=(N_BLOCKS,),
        in_specs=[
            pl.BlockSpec((ROW_BLK, D), lambda i: (i, 0)),
            pl.BlockSpec((NC, ROW_BLK, D), lambda i: (0, i, 0)),
            pl.BlockSpec((D, HID), lambda i: (0, 0)),
            pl.BlockSpec((1, HID), lambda i: (0, 0)),
            pl.BlockSpec((HID, D), lambda i: (0, 0)),
            pl.BlockSpec((1, D), lambda i: (0, 0)),
            pl.BlockSpec((1, 1), lambda i: (0, 0)),
            pl.BlockSpec((D, 1), lambda i: (0, 0)),
            pl.BlockSpec((1, 1, ROW_BLK), lambda i: (i, 0, 0)),
            pl.BlockSpec((D, 1), lambda i: (0, 0)),
            pl.BlockSpec((1, 1), lambda i: (0, 0)),
        ],
        out_specs=[
            pl.BlockSpec((N_GRAPHS, D), lambda i: (0, 0)),
            pl.BlockSpec((N_GRAPHS, 1), lambda i: (0, 0)),
        ],
        out_shape=[
            jax.ShapeDtypeStruct((N_GRAPHS, D), jnp.float32),
            jax.ShapeDtypeStruct((N_GRAPHS, 1), jnp.float32),
        ],
    )(h, parts, w1, b1, w2, b2, epsl, ws_col, batch3, wp_row, bp2)
    del pooled
    return out


def kernel(x, edge_index, batch, W1, b1, W2, b2, eps, w_score, Wp, bp):
    src = edge_index[0]
    dst = edge_index[1]

    srcw = jnp.pad(src.reshape(NW, EDGES_PER_W),
                   ((0, 0), (0, EDGES_PAD - EDGES_PER_W)),
                   constant_values=0).reshape(NW, NCHUNK, CHUNK)
    pad_dst = jnp.broadcast_to(
        DUMMY_ROW + jnp.arange(NW, dtype=jnp.int32)[:, None],
        (NW, EDGES_PAD - EDGES_PER_W))
    dstw = jnp.concatenate(
        [dst.reshape(NW, EDGES_PER_W), pad_dst], axis=1
    ).reshape(NW, NCHUNK, CHUNK)
    zeros = jnp.zeros((ROWS_PAD, D), jnp.float32)
    batch3 = batch.reshape(N_BLOCKS, 1, ROW_BLK)
    wp_row = Wp
    bp2 = bp.reshape(1, 1)

    h = x
    for l in range(NUM_LAYERS):
        parts = _sc_agg(h, srcw, dstw, zeros)
        w1 = W1[l]
        b1l = b1[l].reshape(1, HID)
        w2 = W2[l]
        b2l = b2[l].reshape(1, D)
        epsl = eps[l].reshape(1, 1)
        ws_col = w_score[l].reshape(D, 1)
        if l < NUM_LAYERS - 1:
            h = _tc_mid(h, parts, w1, b1l, w2, b2l, epsl, ws_col)
        else:
            out = _tc_last(h, parts, w1, b1l, w2, b2l, epsl, ws_col,
                           batch3, wp_row, bp2)
    return out
